# baseline (speedup 1.0000x reference)
import functools

import jax
import jax.numpy as jnp
from jax import lax
from jax.experimental import pallas as pl
from jax.experimental.pallas import tpu as pltpu
from jax.experimental.pallas import tpu_sc as plsc

_B = 1024
_S = 2048
_NA = 2048
_H = 4096
_HB = 512
_SB = 256
_BB = 256
_NW = 32
_CHUNK = 8
_NCHUNK = _B // _CHUNK


def _sc_gather_kernel(state, w1, perm, actg, nact, state_c, g_c,
                      permv, actv, sbuf, gbuf, nact_s, sem1, sem2):
    wid = lax.axis_index("s") * 2 + lax.axis_index("c")
    pltpu.sync_copy(nact, nact_s)
    n_act = nact_s[...][0]
    for c in range(_NCHUNK // _NW):
        g = wid + _NW * c
        base = g * _CHUNK

        @pl.when(base < n_act)
        def _do():
            pltpu.sync_copy(perm.at[pl.ds(base, _CHUNK)], permv)
            pltpu.sync_copy(actg.at[pl.ds(base, _CHUNK)], actv)
            cp1 = pltpu.async_copy(state.at[permv], sbuf, sem1)
            cp2 = pltpu.async_copy(w1.at[actv], gbuf, sem2)
            cp1.wait()
            cp2.wait()
            pltpu.sync_copy(sbuf, state_c.at[pl.ds(base, _CHUNK)])
            pltpu.sync_copy(gbuf, g_c.at[pl.ds(base, _CHUNK)])


def _sc_scatter_kernel(ns_c, perm, nact, sct, permv, buf, nact_s, sem):
    wid = lax.axis_index("s") * 2 + lax.axis_index("c")
    pltpu.sync_copy(nact, nact_s)
    n_act = nact_s[...][0]
    for c in range(_NCHUNK // _NW):
        g = wid + _NW * c
        base = g * _CHUNK

        @pl.when(base < n_act)
        def _do():
            pltpu.sync_copy(perm.at[pl.ds(base, _CHUNK)], permv)
            pltpu.sync_copy(ns_c.at[pl.ds(base, _CHUNK)], buf)
            pltpu.async_copy(buf, sct.at[permv], sem).wait()


def _sc_gather(initial_state, W1, perm, act_g, nact_arr):
    mesh = plsc.VectorSubcoreMesh(core_axis_name="c", subcore_axis_name="s")
    fn = functools.partial(
        pl.kernel, mesh=mesh,
        out_type=[
            jax.ShapeDtypeStruct((_B, _S), jnp.float32),
            jax.ShapeDtypeStruct((_B, _H), jnp.float32),
        ],
        scratch_types=[
            pltpu.VMEM((_CHUNK,), jnp.int32),
            pltpu.VMEM((_CHUNK,), jnp.int32),
            pltpu.VMEM((_CHUNK, _S), jnp.float32),
            pltpu.VMEM((_CHUNK, _H), jnp.float32),
            pltpu.VMEM((16,), jnp.int32),
            pltpu.SemaphoreType.DMA,
            pltpu.SemaphoreType.DMA,
        ],
    )(_sc_gather_kernel)
    return fn(initial_state, W1, perm, act_g, nact_arr)


def _sc_scatter(ns_c, perm, nact_arr):
    mesh = plsc.VectorSubcoreMesh(core_axis_name="c", subcore_axis_name="s")
    fn = functools.partial(
        pl.kernel, mesh=mesh,
        out_type=jax.ShapeDtypeStruct((_B, _S), jnp.float32),
        scratch_types=[
            pltpu.VMEM((_CHUNK,), jnp.int32),
            pltpu.VMEM((_CHUNK, _S), jnp.float32),
            pltpu.VMEM((16,), jnp.int32),
            pltpu.SemaphoreType.DMA,
        ],
    )(_sc_scatter_kernel)
    return fn(ns_c, perm, nact_arr)


def _mm1_kernel(nact_ref, state_ref, w1_ref, g_ref, h_ref, x_ref):
    j = pl.program_id(0)
    b = pl.program_id(1)

    @pl.when((j == 0) & (b == 0))
    def _build_x():
        x_ref[...] = state_ref[...].astype(jnp.bfloat16)

    @pl.when(b * _BB < nact_ref[0])
    def _compute():
        xc = x_ref[pl.ds(b * _BB, _BB), :]
        acc = jnp.dot(xc, w1_ref[...].astype(jnp.bfloat16),
                      preferred_element_type=jnp.float32)
        acc = acc + g_ref[...].astype(jnp.bfloat16).astype(jnp.float32)
        h_ref[...] = jnp.maximum(acc, 0.0).astype(jnp.bfloat16)


def _mm2_kernel(nact_ref, h_ref, w2_ref, ns_ref):
    b = pl.program_id(1)

    @pl.when(b * _BB < nact_ref[0])
    def _compute():
        hc = h_ref[pl.ds(b * _BB, _BB), :]
        logits = jnp.dot(hc, w2_ref[...].astype(jnp.bfloat16),
                         preferred_element_type=jnp.float32)
        ns_ref[...] = jax.nn.sigmoid(logits)


def _assemble_kernel(sct_ref, init_ref, out_ref, mask_ref):
    s = pl.program_id(0)

    @pl.when(s == 0)
    def _mask():
        c = init_ref[...]
        in_prog = ((c[:, 1:2] > c[:, 0:1])
                   & (c[:, 2:3] >= c[:, 3:4])
                   & (c[:, 2:3] >= c[:, 4:5]))
        mask_ref[...] = in_prog

    in_prog = mask_ref[...]
    sel = jnp.where(in_prog, sct_ref[...], init_ref[...])
    out_ref[:, pl.ds(s * _SB, _SB)] = sel

    @pl.when(s == 0)
    def _reward():
        step_r = 1000.0 * (sel[:, 14:15] - sel[:, 11:12]
                           + 0.5 * (sel[:, 13:14] - sel[:, 10:11]))
        out_ref[:, _S:] = jnp.where(in_prog, step_r, 0.0)


def _mm1(nact_arr, state_c, W1s, g_c):
    return pl.pallas_call(
        _mm1_kernel,
        grid_spec=pltpu.PrefetchScalarGridSpec(
            num_scalar_prefetch=1,
            grid=(_H // _HB, _B // _BB),
            in_specs=[
                pl.BlockSpec((_B, _S), lambda j, b, n: (0, 0)),
                pl.BlockSpec((_S, _HB), lambda j, b, n: (0, j)),
                pl.BlockSpec((_BB, _HB), lambda j, b, n: (b, j)),
            ],
            out_specs=pl.BlockSpec((_BB, _HB), lambda j, b, n: (b, j)),
            scratch_shapes=[pltpu.VMEM((_B, _S), jnp.bfloat16)],
        ),
        out_shape=jax.ShapeDtypeStruct((_B, _H), jnp.bfloat16),
    )(nact_arr, state_c, W1s, g_c)


def _mm2(nact_arr, h_c, W2):
    return pl.pallas_call(
        _mm2_kernel,
        grid_spec=pltpu.PrefetchScalarGridSpec(
            num_scalar_prefetch=1,
            grid=(_S // _SB, _B // _BB),
            in_specs=[
                pl.BlockSpec((_B, _H), lambda s, b, n: (0, 0)),
                pl.BlockSpec((_H, _SB), lambda s, b, n: (0, s)),
            ],
            out_specs=pl.BlockSpec((_BB, _SB), lambda s, b, n: (b, s)),
        ),
        out_shape=jax.ShapeDtypeStruct((_B, _S), jnp.float32),
    )(nact_arr, h_c, W2)


def _assemble(sct, initial_state):
    return pl.pallas_call(
        _assemble_kernel,
        grid=(_S // _SB,),
        in_specs=[
            pl.BlockSpec((_B, _SB), lambda s: (0, s)),
            pl.BlockSpec((_B, _SB), lambda s: (0, s)),
        ],
        out_specs=pl.BlockSpec((_B, _S + 1), lambda s: (0, 0)),
        out_shape=jax.ShapeDtypeStruct((_B, _S + 1), jnp.float32),
        scratch_shapes=[pltpu.VMEM((_B, 1), jnp.bool_)],
    )(sct, initial_state)


def kernel(initial_state, initial_action, W1, W2, Wa1, Wa2):
    ip = ((initial_state[:, 1] > initial_state[:, 0])
          & (initial_state[:, 2] >= initial_state[:, 3])
          & (initial_state[:, 2] >= initial_state[:, 4]))
    ipi = ip.astype(jnp.int32)
    cs = jnp.cumsum(ipi)
    n_act = cs[-1]
    iota = jnp.arange(_B, dtype=jnp.int32)
    pos = jnp.where(ip, cs - 1, n_act + jnp.cumsum(1 - ipi) - 1)
    perm = jnp.zeros((_B,), jnp.int32).at[pos].set(iota)
    act_g = initial_action[perm].astype(jnp.int32) + _S
    nact_arr = n_act.reshape(1)
    nact16 = jnp.broadcast_to(n_act, (16,))

    state_c, g_c = _sc_gather(initial_state, W1, perm, act_g, nact16)
    h_c = _mm1(nact_arr, state_c, W1[:_S], g_c)
    ns_c = _mm2(nact_arr, h_c, W2)
    sct = _sc_scatter(ns_c, perm, nact16)
    return _assemble(sct, initial_state)


# speedup vs baseline: 1.1944x; 1.1944x over previous
import functools

import jax
import jax.numpy as jnp
from jax import lax
from jax.experimental import pallas as pl
from jax.experimental.pallas import tpu as pltpu
from jax.experimental.pallas import tpu_sc as plsc

_B = 1024
_S = 2048
_NA = 2048
_H = 4096
_HB = 512
_SB = 256
_BB = 256
_NW = 32
_CHUNK = 8
_NCHUNK = _B // _CHUNK


def _sc_gather_kernel(w1, actg, g, actv, gbuf, sem):
    wid = lax.axis_index("s") * 2 + lax.axis_index("c")
    for c in range(_NCHUNK // _NW):
        base = (wid + _NW * c) * _CHUNK
        pltpu.sync_copy(actg.at[pl.ds(base, _CHUNK)], actv)
        pltpu.async_copy(w1.at[actv], gbuf, sem).wait()
        pltpu.sync_copy(gbuf, g.at[pl.ds(base, _CHUNK)])


def _sc_gather(W1, act_g):
    mesh = plsc.VectorSubcoreMesh(core_axis_name="c", subcore_axis_name="s")
    fn = functools.partial(
        pl.kernel, mesh=mesh,
        out_type=jax.ShapeDtypeStruct((_B, _H), jnp.float32),
        scratch_types=[
            pltpu.VMEM((_CHUNK,), jnp.int32),
            pltpu.VMEM((_CHUNK, _H), jnp.float32),
            pltpu.SemaphoreType.DMA,
        ],
    )(_sc_gather_kernel)
    return fn(W1, act_g)


def _mm1_kernel(state_ref, w1_ref, h_ref, x_ref):
    j = pl.program_id(0)
    b = pl.program_id(1)

    @pl.when((j == 0) & (b == 0))
    def _build_x():
        x_ref[...] = state_ref[...].astype(jnp.bfloat16)

    xc = x_ref[pl.ds(b * _BB, _BB), :]
    acc = jnp.dot(xc, w1_ref[...].astype(jnp.bfloat16),
                  preferred_element_type=jnp.float32)
    h_ref[...] = acc


def _mm2_kernel(hpre_ref, g_ref, w2_ref, ns_ref, h_ref):
    s = pl.program_id(0)
    b = pl.program_id(1)

    @pl.when(s == 0)
    def _build_h():
        rows = pl.ds(b * _BB, _BB)
        gb = g_ref[rows, :].astype(jnp.bfloat16).astype(jnp.float32)
        acc = hpre_ref[rows, :] + gb
        h_ref[rows, :] = jnp.maximum(acc, 0.0).astype(jnp.bfloat16)

    hc = h_ref[pl.ds(b * _BB, _BB), :]
    logits = jnp.dot(hc, w2_ref[...].astype(jnp.bfloat16),
                     preferred_element_type=jnp.float32)
    ns_ref[...] = jax.nn.sigmoid(logits)


def _assemble_kernel(ns_ref, init_ref, out_ref, mask_ref):
    s = pl.program_id(0)

    @pl.when(s == 0)
    def _mask():
        c = init_ref[...]
        in_prog = ((c[:, 1:2] > c[:, 0:1])
                   & (c[:, 2:3] >= c[:, 3:4])
                   & (c[:, 2:3] >= c[:, 4:5]))
        mask_ref[...] = in_prog

    in_prog = mask_ref[...]
    sel = jnp.where(in_prog, ns_ref[...], init_ref[...])
    out_ref[:, pl.ds(s * _SB, _SB)] = sel

    @pl.when(s == 0)
    def _reward():
        step_r = 1000.0 * (sel[:, 14:15] - sel[:, 11:12]
                           + 0.5 * (sel[:, 13:14] - sel[:, 10:11]))
        out_ref[:, _S:] = jnp.where(in_prog, step_r, 0.0)


def _mm1(state, W1):
    return pl.pallas_call(
        _mm1_kernel,
        grid=(_H // _HB, _B // _BB),
        in_specs=[
            pl.BlockSpec((_B, _S), lambda j, b: (0, 0)),
            pl.BlockSpec((_S, _HB), lambda j, b: (0, j)),
        ],
        out_specs=pl.BlockSpec((_BB, _HB), lambda j, b: (b, j)),
        out_shape=jax.ShapeDtypeStruct((_B, _H), jnp.float32),
        scratch_shapes=[pltpu.VMEM((_B, _S), jnp.bfloat16)],
    )(state, W1)


def _mm2(hpre, g, W2):
    return pl.pallas_call(
        _mm2_kernel,
        grid=(_S // _SB, _B // _BB),
        in_specs=[
            pl.BlockSpec((_B, _H), lambda s, b: (0, 0)),
            pl.BlockSpec((_B, _H), lambda s, b: (0, 0)),
            pl.BlockSpec((_H, _SB), lambda s, b: (0, s)),
        ],
        out_specs=pl.BlockSpec((_BB, _SB), lambda s, b: (b, s)),
        out_shape=jax.ShapeDtypeStruct((_B, _S), jnp.float32),
        scratch_shapes=[pltpu.VMEM((_B, _H), jnp.bfloat16)],
    )(hpre, g, W2)


def _assemble(ns, initial_state):
    return pl.pallas_call(
        _assemble_kernel,
        grid=(_S // _SB,),
        in_specs=[
            pl.BlockSpec((_B, _SB), lambda s: (0, s)),
            pl.BlockSpec((_B, _SB), lambda s: (0, s)),
        ],
        out_specs=pl.BlockSpec((_B, _S + 1), lambda s: (0, 0)),
        out_shape=jax.ShapeDtypeStruct((_B, _S + 1), jnp.float32),
        scratch_shapes=[pltpu.VMEM((_B, 1), jnp.bool_)],
    )(ns, initial_state)


def kernel(initial_state, initial_action, W1, W2, Wa1, Wa2):
    act_g = initial_action.astype(jnp.int32) + _S
    g = _sc_gather(W1, act_g)
    hpre = _mm1(initial_state, W1)
    ns = _mm2(hpre, g, W2)
    return _assemble(ns, initial_state)
